# Initial kernel scaffold; baseline (speedup 1.0000x reference)
import jax
import jax.numpy as jnp
from jax.experimental import pallas as pl


def kernel(z, W):
    raise NotImplementedError("write your pallas kernel here")


# speedup vs baseline: 1.5948x; 1.5948x over previous
import functools

import jax
import jax.numpy as jnp
from jax import lax
from jax.experimental import pallas as pl
from jax.experimental.pallas import tpu as pltpu
from jax.experimental.pallas import tpu_sc as plsc

_BT = 2048


def _vq_tc_body(nt, d, z_ref, w_ref, idx_ref, loss_ref):
    pid = pl.program_id(0)
    zb = z_ref[...]
    w = w_ref[...]
    k = w.shape[0]
    wsq = jnp.sum(w * w, axis=1)
    zsq = jnp.sum(zb * zb, axis=1)
    mm = lax.dot_general(zb, w, (((1,), (1,)), ((), ())),
                         preferred_element_type=jnp.float32)
    dist = (zsq[:, None] + wsq[None, :]) - 2.0 * mm
    m = jnp.min(dist, axis=1)
    ii = lax.broadcasted_iota(jnp.int32, dist.shape, 1)
    idx = jnp.min(jnp.where(dist == m[:, None], ii, k), axis=1)
    idx_ref[...] = idx

    @pl.when(pid == 0)
    def _init():
        loss_ref[...] = jnp.zeros((1, 1), jnp.float32)

    loss_ref[...] += jnp.sum(m).reshape(1, 1)

    @pl.when(pid == pl.num_programs(0) - 1)
    def _finish():
        loss_ref[...] *= 1.25 / (nt * d)


def _sc_gather(w, idx3, nt, d):
    nw, n_ch, ch = idx3.shape
    b_per_w = n_ch * ch
    info = plsc.get_sparse_core_info()
    mesh = plsc.VectorSubcoreMesh(core_axis_name="c", subcore_axis_name="s")

    @functools.partial(
        pl.kernel, mesh=mesh,
        out_type=jax.ShapeDtypeStruct((nt, d), jnp.float32),
        compiler_params=pltpu.CompilerParams(use_tc_tiling_on_sc=False),
        scratch_types=[
            pltpu.VMEM((n_ch, ch), jnp.int32),
            pltpu.VMEM((b_per_w, d), jnp.float32),
            pltpu.SemaphoreType.DMA,
        ],
    )
    def gk(w_hbm, idx_hbm, out_hbm, idx_v, rows_v, sem):
        wid = lax.axis_index("s") * info.num_cores + lax.axis_index("c")
        base = wid * b_per_w
        pltpu.sync_copy(idx_hbm.at[wid], idx_v)
        copies = [
            pltpu.async_copy(w_hbm.at[idx_v.at[c]],
                             rows_v.at[pl.ds(c * ch, ch)], sem)
            for c in range(n_ch)
        ]
        for cp in copies:
            cp.wait()
        pltpu.sync_copy(rows_v, out_hbm.at[pl.ds(base, b_per_w)])

    return gk(w, idx3)


def kernel(z, W):
    b, s, d = z.shape
    k = W.shape[0]
    z_flat = z.reshape(-1, d)
    nt = z_flat.shape[0]
    n_blk = nt // _BT

    idx, loss_acc = pl.pallas_call(
        functools.partial(_vq_tc_body, nt, d),
        grid=(n_blk,),
        in_specs=[
            pl.BlockSpec((_BT, d), lambda i: (i, 0)),
            pl.BlockSpec((k, d), lambda i: (0, 0)),
        ],
        out_specs=[
            pl.BlockSpec((_BT,), lambda i: (i,)),
            pl.BlockSpec((1, 1), lambda i: (0, 0)),
        ],
        out_shape=[
            jax.ShapeDtypeStruct((nt,), jnp.int32),
            jax.ShapeDtypeStruct((1, 1), jnp.float32),
        ],
    )(z_flat, W)

    nw = 32
    ch = 128
    idx3 = idx.reshape(nw, (nt // nw) // ch, ch)
    q = _sc_gather(W, idx3, nt, d)

    loss = loss_acc[0, 0]
    return q.reshape(z.shape), loss, idx.reshape(b, s)
